# baseline (speedup 1.0000x reference)
import functools

import numpy as np
import jax
import jax.numpy as jnp
from jax import lax
from jax.experimental import pallas as pl
from jax.experimental.pallas import tpu as pltpu
from jax.experimental.pallas import tpu_sc as plsc

_RES = 128
_SIG = 10.0
_ROWS = 8320
_RCHUNK = 320


def _spec_consts():
    freqs = [np.fft.fftfreq(_RES, d=1.0 / _RES)] * 2
    freqs.append(np.fft.rfftfreq(_RES, d=1.0 / _RES))
    om = np.stack(np.meshgrid(*freqs, indexing="ij"), axis=-1)
    dis = np.sqrt((om ** 2).sum(-1))
    g = np.exp(-0.5 * ((_SIG * 2.0 * dis / _RES) ** 2))
    lap = -np.sum((2.0 * np.pi * om) ** 2, axis=-1)
    c = 2.0 * np.pi * g / (lap + 1e-6)
    b = np.moveaxis(om, -1, 0) * c
    return b.astype(np.float32).reshape(3, _ROWS, 128)


_B_CONST = _spec_consts()


def _spectral_combine(Fr, Fi):
    B = jnp.asarray(_B_CONST)

    def body(fr_ref, fi_ref, b_ref, o_ref):
        b0, b1, b2 = b_ref[0], b_ref[1], b_ref[2]
        for b in range(2):
            o_ref[2 * b] = (b0 * fi_ref[3 * b] + b1 * fi_ref[3 * b + 1]
                            + b2 * fi_ref[3 * b + 2])
            o_ref[2 * b + 1] = -(b0 * fr_ref[3 * b] + b1 * fr_ref[3 * b + 1]
                                 + b2 * fr_ref[3 * b + 2])

    return pl.pallas_call(
        body,
        grid=(_ROWS // _RCHUNK,),
        in_specs=[
            pl.BlockSpec((6, _RCHUNK, 128), lambda i: (0, i, 0)),
            pl.BlockSpec((6, _RCHUNK, 128), lambda i: (0, i, 0)),
            pl.BlockSpec((3, _RCHUNK, 128), lambda i: (0, i, 0)),
        ],
        out_specs=pl.BlockSpec((4, _RCHUNK, 128), lambda i: (0, i, 0)),
        out_shape=jax.ShapeDtypeStruct((4, _ROWS, 128), jnp.float32),
    )(Fr, Fi, B)


_P = 100352
_TPTS = _P // 16
_CHUNK = 1568
_NROW = _CHUNK // 16
_SLABW = 64
_SLAB = _SLABW * _RES * _RES
_DUMP = _SLAB
_ACC = _SLAB + 16384
_STRIPE = _SLAB // 16


def _make_sc_rasterize(b, f_feat):
    mesh = plsc.VectorSubcoreMesh(core_axis_name="c", subcore_axis_name="s")
    vbase = b * 3 * _P
    nbase = (b * 3 + f_feat) * _P

    @functools.partial(
        pl.kernel,
        out_type=jax.ShapeDtypeStruct((2 * _SLAB,), jnp.float32),
        mesh=mesh,
        scratch_types=[
            pltpu.VMEM_SHARED((_ACC,), jnp.float32),
            pltpu.VMEM((2048,), jnp.float32),
            pltpu.VMEM((_CHUNK,), jnp.float32),
            pltpu.VMEM((_CHUNK,), jnp.float32),
            pltpu.VMEM((_CHUNK,), jnp.float32),
            pltpu.VMEM((_CHUNK,), jnp.float32),
            pltpu.VMEM((_NROW * 128,), jnp.int32),
            pltpu.VMEM((_NROW * 128,), jnp.float32),
        ],
    )
    def k(v_hbm, n_hbm, out_hbm, acc, zbuf, px, py, pz, nv, idxb, valb):
        slab = lax.axis_index("c")
        s = lax.axis_index("s")

        def zinit(i, carry):
            zbuf[pl.ds(i * 16, 16)] = jnp.zeros((16,), jnp.float32)
            return carry

        lax.fori_loop(0, 2048 // 16, zinit, 0)

        def zero(i, carry2):
            pltpu.sync_copy(zbuf, acc.at[pl.ds(s * _STRIPE + i * 2048, 2048)])
            return carry2

        lax.fori_loop(0, _STRIPE // 2048, zero, 0)
        plsc.subcore_barrier()

        def chunk(cc, carry2):
            pbase = s * _TPTS + cc * _CHUNK
            pltpu.sync_copy(v_hbm.at[pl.ds(vbase + pbase, _CHUNK)], px)
            pltpu.sync_copy(v_hbm.at[pl.ds(vbase + _P + pbase, _CHUNK)], py)
            pltpu.sync_copy(v_hbm.at[pl.ds(vbase + 2 * _P + pbase, _CHUNK)], pz)
            pltpu.sync_copy(n_hbm.at[pl.ds(nbase + pbase, _CHUNK)], nv)

            def row(i, carry3):
                base = i * 16
                tx = px[pl.ds(base, 16)] * 128.0
                x0 = tx.astype(jnp.int32)
                fx = tx - x0.astype(jnp.float32)
                x1 = jnp.where(fx > 0.0, x0 + 1, x0) & 127
                ty = py[pl.ds(base, 16)] * 128.0
                y0 = ty.astype(jnp.int32)
                fy = ty - y0.astype(jnp.float32)
                y1 = jnp.where(fy > 0.0, y0 + 1, y0) & 127
                tz = pz[pl.ds(base, 16)] * 128.0
                z0 = tz.astype(jnp.int32)
                fz = tz - z0.astype(jnp.float32)
                z1 = jnp.where(fz > 0.0, z0 + 1, z0) & 127
                val = nv[pl.ds(base, 16)]

                dump = jnp.full((16,), _DUMP, jnp.int32)
                xo0 = jnp.where((x0 >> 6) == slab, (x0 & 63) * 16384, dump)
                xo1 = jnp.where((x1 >> 6) == slab, (x1 & 63) * 16384, dump)
                a00 = xo0 + y0 * 128
                a01 = xo0 + y1 * 128
                a10 = xo1 + y0 * 128
                a11 = xo1 + y1 * 128
                wx0 = 1.0 - fx
                wy0 = 1.0 - fy
                wz0 = (1.0 - fz) * val
                wz1 = fz * val
                w00 = wx0 * wy0
                w01 = wx0 * fy
                w10 = fx * wy0
                w11 = fx * fy
                idxb[pl.ds(i * 128 + 0, 16)] = a00 + z0
                valb[pl.ds(i * 128 + 0, 16)] = w00 * wz0
                idxb[pl.ds(i * 128 + 16, 16)] = a00 + z1
                valb[pl.ds(i * 128 + 16, 16)] = w00 * wz1
                idxb[pl.ds(i * 128 + 32, 16)] = a01 + z0
                valb[pl.ds(i * 128 + 32, 16)] = w01 * wz0
                idxb[pl.ds(i * 128 + 48, 16)] = a01 + z1
                valb[pl.ds(i * 128 + 48, 16)] = w01 * wz1
                idxb[pl.ds(i * 128 + 64, 16)] = a10 + z0
                valb[pl.ds(i * 128 + 64, 16)] = w10 * wz0
                idxb[pl.ds(i * 128 + 80, 16)] = a10 + z1
                valb[pl.ds(i * 128 + 80, 16)] = w10 * wz1
                idxb[pl.ds(i * 128 + 96, 16)] = a11 + z0
                valb[pl.ds(i * 128 + 96, 16)] = w11 * wz0
                idxb[pl.ds(i * 128 + 112, 16)] = a11 + z1
                valb[pl.ds(i * 128 + 112, 16)] = w11 * wz1
                return carry3

            lax.fori_loop(0, _NROW, row, 0)
            pltpu.sync_copy(valb, acc.at[idxb], add=True)
            return carry2

        lax.fori_loop(0, _TPTS // _CHUNK, chunk, 0)
        plsc.subcore_barrier()

        pltpu.sync_copy(
            acc.at[pl.ds(s * _STRIPE, _STRIPE)],
            out_hbm.at[pl.ds(slab * _SLAB + s * _STRIPE, _STRIPE)])

    return k


_SC_RASTER = [[_make_sc_rasterize(b, f) for f in range(3)] for b in range(2)]


def kernel(V, N):
    npts = V.shape[1]
    Vt = jnp.pad(jnp.transpose(V, (0, 2, 1)),
                 ((0, 0), (0, 0), (0, _P - npts))).reshape(-1)
    Nt = jnp.pad(jnp.transpose(N, (0, 2, 1)),
                 ((0, 0), (0, 0), (0, _P - npts))).reshape(-1)
    Fs = []
    for b in range(2):
        for f in range(3):
            g = _SC_RASTER[b][f](Vt, Nt)
            Fs.append(jnp.fft.rfftn(g.reshape(_RES, _RES, _RES)))
    F = jnp.stack(Fs)
    Fr = jnp.real(F).reshape(6, _ROWS, 128)
    Fi = jnp.imag(F).reshape(6, _ROWS, 128)
    O = _spectral_combine(Fr, Fi)
    Phi = O.reshape(2, 2, 128, 128, 65).transpose(2, 3, 4, 1, 0)
    return Phi.at[0, 0, 0].set(0.0)


# speedup vs baseline: 1.3514x; 1.3514x over previous
import functools

import numpy as np
import jax
import jax.numpy as jnp
from jax import lax
from jax.experimental import pallas as pl
from jax.experimental.pallas import tpu as pltpu
from jax.experimental.pallas import tpu_sc as plsc

_RES = 128
_SIG = 10.0
_ROWS = 8320
_RCHUNK = 320


def _spec_consts():
    freqs = [np.fft.fftfreq(_RES, d=1.0 / _RES)] * 2
    freqs.append(np.fft.rfftfreq(_RES, d=1.0 / _RES))
    om = np.stack(np.meshgrid(*freqs, indexing="ij"), axis=-1)
    dis = np.sqrt((om ** 2).sum(-1))
    g = np.exp(-0.5 * ((_SIG * 2.0 * dis / _RES) ** 2))
    lap = -np.sum((2.0 * np.pi * om) ** 2, axis=-1)
    a = g / (lap + 1e-6)
    return a.astype(np.float32).reshape(_ROWS, 128)


def _deriv_matrix():
    om = np.fft.fftfreq(_RES, d=1.0 / _RES)
    h = -2j * np.pi * om
    h[_RES // 2] = 0.0
    c = np.real(np.fft.ifft(h))
    idx = (np.arange(_RES)[:, None] - np.arange(_RES)[None, :]) % _RES
    return c[idx].astype(np.float32)


_A_CONST = _spec_consts()
_M_CONST = _deriv_matrix()


def _conv_div(g0, g1, g2):
    M = jnp.asarray(_M_CONST)
    MT = jnp.asarray(np.ascontiguousarray(_M_CONST.T))

    def body(g0_ref, g1_ref, g2_ref, m_ref, mt_ref, o_ref):
        j = pl.program_id(0)
        m = m_ref[...]
        mt = mt_ref[...]
        mrows = m_ref[pl.ds(j * 8, 8), :]
        g0r = g0_ref[...].reshape(_RES, _RES * _RES)
        xterm = jnp.dot(mrows, g0r, preferred_element_type=jnp.float32,
                        precision=jax.lax.Precision.HIGHEST)
        o_ref[...] = xterm.reshape(8, _RES, _RES)
        for p in range(8):
            yterm = jnp.dot(m, g1_ref[p], preferred_element_type=jnp.float32,
                            precision=jax.lax.Precision.HIGHEST)
            zterm = jnp.dot(g2_ref[p], mt, preferred_element_type=jnp.float32,
                            precision=jax.lax.Precision.HIGHEST)
            o_ref[p] += yterm + zterm

    return pl.pallas_call(
        body,
        grid=(_RES // 8,),
        in_specs=[
            pl.BlockSpec((_RES, _RES, _RES), lambda j: (0, 0, 0)),
            pl.BlockSpec((8, _RES, _RES), lambda j: (j, 0, 0)),
            pl.BlockSpec((8, _RES, _RES), lambda j: (j, 0, 0)),
            pl.BlockSpec((_RES, _RES), lambda j: (0, 0)),
            pl.BlockSpec((_RES, _RES), lambda j: (0, 0)),
        ],
        out_specs=pl.BlockSpec((8, _RES, _RES), lambda j: (j, 0, 0)),
        out_shape=jax.ShapeDtypeStruct((_RES, _RES, _RES), jnp.float32),
    )(g0, g1, g2, M, MT)


def _scale_combine(Sr, Si):
    A = jnp.asarray(_A_CONST)

    def body(sr_ref, si_ref, a_ref, o_ref):
        a = a_ref[...]
        for b in range(2):
            o_ref[2 * b] = a * sr_ref[b]
            o_ref[2 * b + 1] = a * si_ref[b]

    return pl.pallas_call(
        body,
        grid=(_ROWS // _RCHUNK,),
        in_specs=[
            pl.BlockSpec((2, _RCHUNK, 128), lambda i: (0, i, 0)),
            pl.BlockSpec((2, _RCHUNK, 128), lambda i: (0, i, 0)),
            pl.BlockSpec((_RCHUNK, 128), lambda i: (i, 0)),
        ],
        out_specs=pl.BlockSpec((4, _RCHUNK, 128), lambda i: (0, i, 0)),
        out_shape=jax.ShapeDtypeStruct((4, _ROWS, 128), jnp.float32),
    )(Sr, Si, A)


_P = 100352
_TPTS = _P // 16
_CHUNK = 1568
_NROW = _CHUNK // 16
_SLABW = 64
_SLAB = _SLABW * _RES * _RES
_DUMP = _SLAB
_ACC = _SLAB + 16384
_STRIPE = _SLAB // 16


def _make_sc_rasterize(b, f_feat):
    mesh = plsc.VectorSubcoreMesh(core_axis_name="c", subcore_axis_name="s")
    vbase = b * 3 * _P
    nbase = (b * 3 + f_feat) * _P

    @functools.partial(
        pl.kernel,
        out_type=jax.ShapeDtypeStruct((2 * _SLAB,), jnp.float32),
        mesh=mesh,
        scratch_types=[
            pltpu.VMEM_SHARED((_ACC,), jnp.float32),
            pltpu.VMEM((2048,), jnp.float32),
            pltpu.VMEM((_CHUNK,), jnp.float32),
            pltpu.VMEM((_CHUNK,), jnp.float32),
            pltpu.VMEM((_CHUNK,), jnp.float32),
            pltpu.VMEM((_CHUNK,), jnp.float32),
            pltpu.VMEM((_NROW * 128,), jnp.int32),
            pltpu.VMEM((_NROW * 128,), jnp.float32),
        ],
    )
    def k(v_hbm, n_hbm, out_hbm, acc, zbuf, px, py, pz, nv, idxb, valb):
        slab = lax.axis_index("c")
        s = lax.axis_index("s")

        def zinit(i, carry):
            zbuf[pl.ds(i * 16, 16)] = jnp.zeros((16,), jnp.float32)
            return carry

        lax.fori_loop(0, 2048 // 16, zinit, 0)

        def zero(i, carry2):
            pltpu.sync_copy(zbuf, acc.at[pl.ds(s * _STRIPE + i * 2048, 2048)])
            return carry2

        lax.fori_loop(0, _STRIPE // 2048, zero, 0)
        plsc.subcore_barrier()

        def chunk(cc, carry2):
            pbase = s * _TPTS + cc * _CHUNK
            pltpu.sync_copy(v_hbm.at[pl.ds(vbase + pbase, _CHUNK)], px)
            pltpu.sync_copy(v_hbm.at[pl.ds(vbase + _P + pbase, _CHUNK)], py)
            pltpu.sync_copy(v_hbm.at[pl.ds(vbase + 2 * _P + pbase, _CHUNK)], pz)
            pltpu.sync_copy(n_hbm.at[pl.ds(nbase + pbase, _CHUNK)], nv)

            def row(i, carry3):
                base = i * 16
                tx = px[pl.ds(base, 16)] * 128.0
                x0 = tx.astype(jnp.int32)
                fx = tx - x0.astype(jnp.float32)
                x1 = jnp.where(fx > 0.0, x0 + 1, x0) & 127
                ty = py[pl.ds(base, 16)] * 128.0
                y0 = ty.astype(jnp.int32)
                fy = ty - y0.astype(jnp.float32)
                y1 = jnp.where(fy > 0.0, y0 + 1, y0) & 127
                tz = pz[pl.ds(base, 16)] * 128.0
                z0 = tz.astype(jnp.int32)
                fz = tz - z0.astype(jnp.float32)
                z1 = jnp.where(fz > 0.0, z0 + 1, z0) & 127
                val = nv[pl.ds(base, 16)]

                dump = jnp.full((16,), _DUMP, jnp.int32)
                xo0 = jnp.where((x0 >> 6) == slab, (x0 & 63) * 16384, dump)
                xo1 = jnp.where((x1 >> 6) == slab, (x1 & 63) * 16384, dump)
                a00 = xo0 + y0 * 128
                a01 = xo0 + y1 * 128
                a10 = xo1 + y0 * 128
                a11 = xo1 + y1 * 128
                wx0 = 1.0 - fx
                wy0 = 1.0 - fy
                wz0 = (1.0 - fz) * val
                wz1 = fz * val
                w00 = wx0 * wy0
                w01 = wx0 * fy
                w10 = fx * wy0
                w11 = fx * fy
                idxb[pl.ds(i * 128 + 0, 16)] = a00 + z0
                valb[pl.ds(i * 128 + 0, 16)] = w00 * wz0
                idxb[pl.ds(i * 128 + 16, 16)] = a00 + z1
                valb[pl.ds(i * 128 + 16, 16)] = w00 * wz1
                idxb[pl.ds(i * 128 + 32, 16)] = a01 + z0
                valb[pl.ds(i * 128 + 32, 16)] = w01 * wz0
                idxb[pl.ds(i * 128 + 48, 16)] = a01 + z1
                valb[pl.ds(i * 128 + 48, 16)] = w01 * wz1
                idxb[pl.ds(i * 128 + 64, 16)] = a10 + z0
                valb[pl.ds(i * 128 + 64, 16)] = w10 * wz0
                idxb[pl.ds(i * 128 + 80, 16)] = a10 + z1
                valb[pl.ds(i * 128 + 80, 16)] = w10 * wz1
                idxb[pl.ds(i * 128 + 96, 16)] = a11 + z0
                valb[pl.ds(i * 128 + 96, 16)] = w11 * wz0
                idxb[pl.ds(i * 128 + 112, 16)] = a11 + z1
                valb[pl.ds(i * 128 + 112, 16)] = w11 * wz1
                return carry3

            lax.fori_loop(0, _NROW, row, 0)
            pltpu.sync_copy(valb, acc.at[idxb], add=True)
            return carry2

        lax.fori_loop(0, _TPTS // _CHUNK, chunk, 0)
        plsc.subcore_barrier()

        pltpu.sync_copy(
            acc.at[pl.ds(s * _STRIPE, _STRIPE)],
            out_hbm.at[pl.ds(slab * _SLAB + s * _STRIPE, _STRIPE)])

    return k


_SC_RASTER = [[_make_sc_rasterize(b, f) for f in range(3)] for b in range(2)]


def kernel(V, N):
    npts = V.shape[1]
    Vt = jnp.pad(jnp.transpose(V, (0, 2, 1)),
                 ((0, 0), (0, 0), (0, _P - npts))).reshape(-1)
    Nt = jnp.pad(jnp.transpose(N, (0, 2, 1)),
                 ((0, 0), (0, 0), (0, _P - npts))).reshape(-1)
    us = []
    for b in range(2):
        g = [_SC_RASTER[b][f](Vt, Nt).reshape(_RES, _RES, _RES)
             for f in range(3)]
        us.append(_conv_div(g[0], g[1], g[2]))
    u = jnp.stack(us)
    S = jnp.fft.rfftn(u, axes=(1, 2, 3))
    Sr = jnp.real(S).reshape(2, _ROWS, 128)
    Si = jnp.imag(S).reshape(2, _ROWS, 128)
    O = _scale_combine(Sr, Si)
    Phi = O.reshape(2, 2, 128, 128, 65).transpose(2, 3, 4, 1, 0)
    return Phi.at[0, 0, 0].set(0.0)


# speedup vs baseline: 1.3536x; 1.0016x over previous
import functools

import numpy as np
import jax
import jax.numpy as jnp
from jax import lax
from jax.experimental import pallas as pl
from jax.experimental.pallas import tpu as pltpu
from jax.experimental.pallas import tpu_sc as plsc

_RES = 128
_SIG = 10.0
_ROWS = 8320
_RCHUNK = 320


def _spec_consts():
    freqs = [np.fft.fftfreq(_RES, d=1.0 / _RES)] * 2
    freqs.append(np.fft.rfftfreq(_RES, d=1.0 / _RES))
    om = np.stack(np.meshgrid(*freqs, indexing="ij"), axis=-1)
    dis = np.sqrt((om ** 2).sum(-1))
    g = np.exp(-0.5 * ((_SIG * 2.0 * dis / _RES) ** 2))
    lap = -np.sum((2.0 * np.pi * om) ** 2, axis=-1)
    a = g / (lap + 1e-6)
    a[0, 0, 0] = 0.0
    return a.astype(np.float32).reshape(_ROWS, 128)


def _deriv_matrix():
    om = np.fft.fftfreq(_RES, d=1.0 / _RES)
    h = -2j * np.pi * om
    h[_RES // 2] = 0.0
    c = np.real(np.fft.ifft(h))
    idx = (np.arange(_RES)[:, None] - np.arange(_RES)[None, :]) % _RES
    return c[idx].astype(np.float32)


_A_CONST = _spec_consts()
_M_CONST = _deriv_matrix()


def _conv_div(g0, g1, g2):
    M = jnp.asarray(_M_CONST)
    MT = jnp.asarray(np.ascontiguousarray(_M_CONST.T))

    def body(g0_ref, g1_ref, g2_ref, m_ref, mt_ref, o_ref):
        j = pl.program_id(0)
        m = m_ref[...]
        mt = mt_ref[...]
        mrows = m_ref[pl.ds(j * 8, 8), :]
        g0r = g0_ref[...].reshape(_RES, _RES * _RES)
        xterm = jnp.dot(mrows, g0r, preferred_element_type=jnp.float32,
                        precision=jax.lax.Precision.HIGHEST)
        o_ref[...] = xterm.reshape(8, _RES, _RES)
        for p in range(8):
            yterm = jnp.dot(m, g1_ref[p], preferred_element_type=jnp.float32,
                            precision=jax.lax.Precision.HIGHEST)
            zterm = jnp.dot(g2_ref[p], mt, preferred_element_type=jnp.float32,
                            precision=jax.lax.Precision.HIGHEST)
            o_ref[p] += yterm + zterm

    return pl.pallas_call(
        body,
        grid=(_RES // 8,),
        in_specs=[
            pl.BlockSpec((_RES, _RES, _RES), lambda j: (0, 0, 0)),
            pl.BlockSpec((8, _RES, _RES), lambda j: (j, 0, 0)),
            pl.BlockSpec((8, _RES, _RES), lambda j: (j, 0, 0)),
            pl.BlockSpec((_RES, _RES), lambda j: (0, 0)),
            pl.BlockSpec((_RES, _RES), lambda j: (0, 0)),
        ],
        out_specs=pl.BlockSpec((8, _RES, _RES), lambda j: (j, 0, 0)),
        out_shape=jax.ShapeDtypeStruct((_RES, _RES, _RES), jnp.float32),
    )(g0, g1, g2, M, MT)


def _scale_combine(Sr, Si):
    A = jnp.asarray(_A_CONST)

    def body(sr_ref, si_ref, a_ref, o_ref):
        a = a_ref[...]
        for b in range(2):
            o_ref[2 * b] = a * sr_ref[b]
            o_ref[2 * b + 1] = a * si_ref[b]

    return pl.pallas_call(
        body,
        grid=(_ROWS // _RCHUNK,),
        in_specs=[
            pl.BlockSpec((2, _RCHUNK, 128), lambda i: (0, i, 0)),
            pl.BlockSpec((2, _RCHUNK, 128), lambda i: (0, i, 0)),
            pl.BlockSpec((_RCHUNK, 128), lambda i: (i, 0)),
        ],
        out_specs=pl.BlockSpec((4, _RCHUNK, 128), lambda i: (0, i, 0)),
        out_shape=jax.ShapeDtypeStruct((4, _ROWS, 128), jnp.float32),
    )(Sr, Si, A)


_P = 100352
_TPTS = _P // 16
_CHUNK = 1568
_NROW = _CHUNK // 16
_SLABW = 64
_SLAB = _SLABW * _RES * _RES
_DUMP = _SLAB
_ACC = _SLAB + 16384
_STRIPE = _SLAB // 16


def _make_sc_rasterize(b, f_feat):
    mesh = plsc.VectorSubcoreMesh(core_axis_name="c", subcore_axis_name="s")
    vbase = b * 3 * _P
    nbase = (b * 3 + f_feat) * _P

    @functools.partial(
        pl.kernel,
        out_type=jax.ShapeDtypeStruct((2 * _SLAB,), jnp.float32),
        mesh=mesh,
        scratch_types=[
            pltpu.VMEM_SHARED((_ACC,), jnp.float32),
            pltpu.VMEM((2048,), jnp.float32),
            pltpu.VMEM((_CHUNK,), jnp.float32),
            pltpu.VMEM((_CHUNK,), jnp.float32),
            pltpu.VMEM((_CHUNK,), jnp.float32),
            pltpu.VMEM((_CHUNK,), jnp.float32),
            pltpu.VMEM((_NROW * 128,), jnp.int32),
            pltpu.VMEM((_NROW * 128,), jnp.float32),
        ],
    )
    def k(v_hbm, n_hbm, out_hbm, acc, zbuf, px, py, pz, nv, idxb, valb):
        slab = lax.axis_index("c")
        s = lax.axis_index("s")

        def zinit(i, carry):
            zbuf[pl.ds(i * 16, 16)] = jnp.zeros((16,), jnp.float32)
            return carry

        lax.fori_loop(0, 2048 // 16, zinit, 0)

        def zero(i, carry2):
            pltpu.sync_copy(zbuf, acc.at[pl.ds(s * _STRIPE + i * 2048, 2048)])
            return carry2

        lax.fori_loop(0, _STRIPE // 2048, zero, 0)
        plsc.subcore_barrier()

        def chunk(cc, carry2):
            pbase = s * _TPTS + cc * _CHUNK
            pltpu.sync_copy(v_hbm.at[pl.ds(vbase + pbase, _CHUNK)], px)
            pltpu.sync_copy(v_hbm.at[pl.ds(vbase + _P + pbase, _CHUNK)], py)
            pltpu.sync_copy(v_hbm.at[pl.ds(vbase + 2 * _P + pbase, _CHUNK)], pz)
            pltpu.sync_copy(n_hbm.at[pl.ds(nbase + pbase, _CHUNK)], nv)

            def row(i, carry3):
                base = i * 16
                tx = px[pl.ds(base, 16)] * 128.0
                x0 = tx.astype(jnp.int32)
                fx = tx - x0.astype(jnp.float32)
                x1 = jnp.where(fx > 0.0, x0 + 1, x0) & 127
                ty = py[pl.ds(base, 16)] * 128.0
                y0 = ty.astype(jnp.int32)
                fy = ty - y0.astype(jnp.float32)
                y1 = jnp.where(fy > 0.0, y0 + 1, y0) & 127
                tz = pz[pl.ds(base, 16)] * 128.0
                z0 = tz.astype(jnp.int32)
                fz = tz - z0.astype(jnp.float32)
                z1 = jnp.where(fz > 0.0, z0 + 1, z0) & 127
                val = nv[pl.ds(base, 16)]

                dump = jnp.full((16,), _DUMP, jnp.int32)
                xo0 = jnp.where((x0 >> 6) == slab, (x0 & 63) * 16384, dump)
                xo1 = jnp.where((x1 >> 6) == slab, (x1 & 63) * 16384, dump)
                a00 = xo0 + y0 * 128
                a01 = xo0 + y1 * 128
                a10 = xo1 + y0 * 128
                a11 = xo1 + y1 * 128
                wx0 = 1.0 - fx
                wy0 = 1.0 - fy
                wz0 = (1.0 - fz) * val
                wz1 = fz * val
                w00 = wx0 * wy0
                w01 = wx0 * fy
                w10 = fx * wy0
                w11 = fx * fy
                idxb[pl.ds(i * 128 + 0, 16)] = a00 + z0
                valb[pl.ds(i * 128 + 0, 16)] = w00 * wz0
                idxb[pl.ds(i * 128 + 16, 16)] = a00 + z1
                valb[pl.ds(i * 128 + 16, 16)] = w00 * wz1
                idxb[pl.ds(i * 128 + 32, 16)] = a01 + z0
                valb[pl.ds(i * 128 + 32, 16)] = w01 * wz0
                idxb[pl.ds(i * 128 + 48, 16)] = a01 + z1
                valb[pl.ds(i * 128 + 48, 16)] = w01 * wz1
                idxb[pl.ds(i * 128 + 64, 16)] = a10 + z0
                valb[pl.ds(i * 128 + 64, 16)] = w10 * wz0
                idxb[pl.ds(i * 128 + 80, 16)] = a10 + z1
                valb[pl.ds(i * 128 + 80, 16)] = w10 * wz1
                idxb[pl.ds(i * 128 + 96, 16)] = a11 + z0
                valb[pl.ds(i * 128 + 96, 16)] = w11 * wz0
                idxb[pl.ds(i * 128 + 112, 16)] = a11 + z1
                valb[pl.ds(i * 128 + 112, 16)] = w11 * wz1
                return carry3

            lax.fori_loop(0, _NROW, row, 0)
            pltpu.sync_copy(valb, acc.at[idxb], add=True)
            return carry2

        lax.fori_loop(0, _TPTS // _CHUNK, chunk, 0)
        plsc.subcore_barrier()

        pltpu.sync_copy(
            acc.at[pl.ds(s * _STRIPE, _STRIPE)],
            out_hbm.at[pl.ds(slab * _SLAB + s * _STRIPE, _STRIPE)])

    return k


_SC_RASTER = [[_make_sc_rasterize(b, f) for f in range(3)] for b in range(2)]


def kernel(V, N):
    npts = V.shape[1]
    Vt = jnp.pad(jnp.transpose(V, (0, 2, 1)),
                 ((0, 0), (0, 0), (0, _P - npts))).reshape(-1)
    Nt = jnp.pad(jnp.transpose(N, (0, 2, 1)),
                 ((0, 0), (0, 0), (0, _P - npts))).reshape(-1)
    us = []
    for b in range(2):
        g = [_SC_RASTER[b][f](Vt, Nt).reshape(_RES, _RES, _RES)
             for f in range(3)]
        us.append(_conv_div(g[0], g[1], g[2]))
    u = jnp.stack(us)
    S = jnp.fft.rfftn(u, axes=(1, 2, 3))
    Sr = jnp.real(S).reshape(2, _ROWS, 128)
    Si = jnp.imag(S).reshape(2, _ROWS, 128)
    O = _scale_combine(Sr, Si)
    return O.reshape(2, 2, 128, 128, 65).transpose(2, 3, 4, 1, 0)


# speedup vs baseline: 1.4878x; 1.0992x over previous
import functools

import numpy as np
import jax
import jax.numpy as jnp
from jax import lax
from jax.experimental import pallas as pl
from jax.experimental.pallas import tpu as pltpu
from jax.experimental.pallas import tpu_sc as plsc

_RES = 128
_SIG = 10.0
_ROWS = 8320
_RCHUNK = 320


def _spec_consts():
    freqs = [np.fft.fftfreq(_RES, d=1.0 / _RES)] * 2
    freqs.append(np.fft.rfftfreq(_RES, d=1.0 / _RES))
    om = np.stack(np.meshgrid(*freqs, indexing="ij"), axis=-1)
    dis = np.sqrt((om ** 2).sum(-1))
    g = np.exp(-0.5 * ((_SIG * 2.0 * dis / _RES) ** 2))
    lap = -np.sum((2.0 * np.pi * om) ** 2, axis=-1)
    a = g / (lap + 1e-6)
    a[0, 0, 0] = 0.0
    return a.astype(np.float32).reshape(_ROWS, 128)


def _deriv_matrix():
    om = np.fft.fftfreq(_RES, d=1.0 / _RES)
    h = -2j * np.pi * om
    h[_RES // 2] = 0.0
    c = np.real(np.fft.ifft(h))
    idx = (np.arange(_RES)[:, None] - np.arange(_RES)[None, :]) % _RES
    return c[idx].astype(np.float32)


_A_CONST = _spec_consts()
_M_CONST = _deriv_matrix()


def _conv_div(g0, g1, g2):
    M = jnp.asarray(_M_CONST)
    MT = jnp.asarray(np.ascontiguousarray(_M_CONST.T))

    def body(g0_ref, g1_ref, g2_ref, m_ref, mt_ref, o_ref):
        j = pl.program_id(0)
        m = m_ref[...]
        mt = mt_ref[...]
        mrows = m_ref[pl.ds(j * 8, 8), :]
        g0r = g0_ref[...].reshape(_RES, _RES * _RES)
        xterm = jnp.dot(mrows, g0r, preferred_element_type=jnp.float32,
                        precision=jax.lax.Precision.HIGHEST)
        o_ref[...] = xterm.reshape(8, _RES, _RES)
        for p in range(8):
            yterm = jnp.dot(m, g1_ref[p], preferred_element_type=jnp.float32,
                            precision=jax.lax.Precision.HIGHEST)
            zterm = jnp.dot(g2_ref[p], mt, preferred_element_type=jnp.float32,
                            precision=jax.lax.Precision.HIGHEST)
            o_ref[p] += yterm + zterm

    return pl.pallas_call(
        body,
        grid=(_RES // 8,),
        in_specs=[
            pl.BlockSpec((_RES, _RES, _RES), lambda j: (0, 0, 0)),
            pl.BlockSpec((8, _RES, _RES), lambda j: (j, 0, 0)),
            pl.BlockSpec((8, _RES, _RES), lambda j: (j, 0, 0)),
            pl.BlockSpec((_RES, _RES), lambda j: (0, 0)),
            pl.BlockSpec((_RES, _RES), lambda j: (0, 0)),
        ],
        out_specs=pl.BlockSpec((8, _RES, _RES), lambda j: (j, 0, 0)),
        out_shape=jax.ShapeDtypeStruct((_RES, _RES, _RES), jnp.float32),
    )(g0, g1, g2, M, MT)


def _scale_combine(Sr, Si):
    A = jnp.asarray(_A_CONST)

    def body(sr_ref, si_ref, a_ref, o_ref):
        a = a_ref[...]
        for b in range(2):
            o_ref[2 * b] = a * sr_ref[b]
            o_ref[2 * b + 1] = a * si_ref[b]

    return pl.pallas_call(
        body,
        grid=(_ROWS // _RCHUNK,),
        in_specs=[
            pl.BlockSpec((2, _RCHUNK, 128), lambda i: (0, i, 0)),
            pl.BlockSpec((2, _RCHUNK, 128), lambda i: (0, i, 0)),
            pl.BlockSpec((_RCHUNK, 128), lambda i: (i, 0)),
        ],
        out_specs=pl.BlockSpec((4, _RCHUNK, 128), lambda i: (0, i, 0)),
        out_shape=jax.ShapeDtypeStruct((4, _ROWS, 128), jnp.float32),
    )(Sr, Si, A)


_P = 100352
_TPTS = _P // 16
_CHUNK = 1568
_NROW = _CHUNK // 16
_SLABW = 64
_SLAB = _SLABW * _RES * _RES
_DUMP = _SLAB
_ACC = _SLAB + 16384
_STRIPE = _SLAB // 16


def _make_sc_rasterize(b, f_feat):
    mesh = plsc.VectorSubcoreMesh(core_axis_name="c", subcore_axis_name="s")
    vbase = b * 3 * _P
    nbase = (b * 3 + f_feat) * _P

    @functools.partial(
        pl.kernel,
        out_type=jax.ShapeDtypeStruct((2 * _SLAB,), jnp.float32),
        mesh=mesh,
        scratch_types=[
            pltpu.VMEM_SHARED((_ACC,), jnp.float32),
            pltpu.VMEM((2048,), jnp.float32),
            pltpu.VMEM((_CHUNK,), jnp.float32),
            pltpu.VMEM((_CHUNK,), jnp.float32),
            pltpu.VMEM((_CHUNK,), jnp.float32),
            pltpu.VMEM((_CHUNK,), jnp.float32),
            pltpu.VMEM((_NROW * 128,), jnp.int32),
            pltpu.VMEM((_NROW * 128,), jnp.float32),
            pltpu.VMEM((_NROW * 128,), jnp.int32),
            pltpu.VMEM((_NROW * 128,), jnp.float32),
            pltpu.SemaphoreType.DMA,
            pltpu.SemaphoreType.DMA,
        ],
    )
    def k(v_hbm, n_hbm, out_hbm, acc, zbuf, px, py, pz, nv,
          idxb0, valb0, idxb1, valb1, sem0, sem1):
        slab = lax.axis_index("c")
        s = lax.axis_index("s")

        def zinit(i, carry):
            zbuf[pl.ds(i * 16, 16)] = jnp.zeros((16,), jnp.float32)
            return carry

        lax.fori_loop(0, 2048 // 16, zinit, 0)

        def zero(i, carry2):
            pltpu.sync_copy(zbuf, acc.at[pl.ds(s * _STRIPE + i * 2048, 2048)])
            return carry2

        lax.fori_loop(0, _STRIPE // 2048, zero, 0)
        plsc.subcore_barrier()

        def chunk(cc, idxb, valb, sem):
            pbase = s * _TPTS + cc * _CHUNK
            pltpu.sync_copy(v_hbm.at[pl.ds(vbase + pbase, _CHUNK)], px)
            pltpu.sync_copy(v_hbm.at[pl.ds(vbase + _P + pbase, _CHUNK)], py)
            pltpu.sync_copy(v_hbm.at[pl.ds(vbase + 2 * _P + pbase, _CHUNK)], pz)
            pltpu.sync_copy(n_hbm.at[pl.ds(nbase + pbase, _CHUNK)], nv)

            def row(i, carry3):
                base = i * 16
                tx = px[pl.ds(base, 16)] * 128.0
                x0 = tx.astype(jnp.int32)
                fx = tx - x0.astype(jnp.float32)
                x1 = jnp.where(fx > 0.0, x0 + 1, x0) & 127
                ty = py[pl.ds(base, 16)] * 128.0
                y0 = ty.astype(jnp.int32)
                fy = ty - y0.astype(jnp.float32)
                y1 = jnp.where(fy > 0.0, y0 + 1, y0) & 127
                tz = pz[pl.ds(base, 16)] * 128.0
                z0 = tz.astype(jnp.int32)
                fz = tz - z0.astype(jnp.float32)
                z1 = jnp.where(fz > 0.0, z0 + 1, z0) & 127
                val = nv[pl.ds(base, 16)]

                dump = jnp.full((16,), _DUMP, jnp.int32)
                xo0 = jnp.where((x0 >> 6) == slab, (x0 & 63) * 16384, dump)
                xo1 = jnp.where((x1 >> 6) == slab, (x1 & 63) * 16384, dump)
                a00 = xo0 + y0 * 128
                a01 = xo0 + y1 * 128
                a10 = xo1 + y0 * 128
                a11 = xo1 + y1 * 128
                wx0 = 1.0 - fx
                wy0 = 1.0 - fy
                wz0 = (1.0 - fz) * val
                wz1 = fz * val
                w00 = wx0 * wy0
                w01 = wx0 * fy
                w10 = fx * wy0
                w11 = fx * fy
                idxb[pl.ds(i * 128 + 0, 16)] = a00 + z0
                valb[pl.ds(i * 128 + 0, 16)] = w00 * wz0
                idxb[pl.ds(i * 128 + 16, 16)] = a00 + z1
                valb[pl.ds(i * 128 + 16, 16)] = w00 * wz1
                idxb[pl.ds(i * 128 + 32, 16)] = a01 + z0
                valb[pl.ds(i * 128 + 32, 16)] = w01 * wz0
                idxb[pl.ds(i * 128 + 48, 16)] = a01 + z1
                valb[pl.ds(i * 128 + 48, 16)] = w01 * wz1
                idxb[pl.ds(i * 128 + 64, 16)] = a10 + z0
                valb[pl.ds(i * 128 + 64, 16)] = w10 * wz0
                idxb[pl.ds(i * 128 + 80, 16)] = a10 + z1
                valb[pl.ds(i * 128 + 80, 16)] = w10 * wz1
                idxb[pl.ds(i * 128 + 96, 16)] = a11 + z0
                valb[pl.ds(i * 128 + 96, 16)] = w11 * wz0
                idxb[pl.ds(i * 128 + 112, 16)] = a11 + z1
                valb[pl.ds(i * 128 + 112, 16)] = w11 * wz1
                return carry3

            lax.fori_loop(0, _NROW, row, 0)
            return pltpu.async_copy(valb, acc.at[idxb], sem, add=True)

        bufs = ((idxb0, valb0, sem0), (idxb1, valb1, sem1))
        handles = [None, None]
        for cc in range(_TPTS // _CHUNK):
            sel = cc % 2
            if handles[sel] is not None:
                handles[sel].wait()
            handles[sel] = chunk(cc, *bufs[sel])
        for h in handles:
            h.wait()
        plsc.subcore_barrier()

        pltpu.sync_copy(
            acc.at[pl.ds(s * _STRIPE, _STRIPE)],
            out_hbm.at[pl.ds(slab * _SLAB + s * _STRIPE, _STRIPE)])

    return k


_SC_RASTER = [[_make_sc_rasterize(b, f) for f in range(3)] for b in range(2)]


def kernel(V, N):
    npts = V.shape[1]
    Vt = jnp.pad(jnp.transpose(V, (0, 2, 1)),
                 ((0, 0), (0, 0), (0, _P - npts))).reshape(-1)
    Nt = jnp.pad(jnp.transpose(N, (0, 2, 1)),
                 ((0, 0), (0, 0), (0, _P - npts))).reshape(-1)
    us = []
    for b in range(2):
        g = [_SC_RASTER[b][f](Vt, Nt).reshape(_RES, _RES, _RES)
             for f in range(3)]
        us.append(_conv_div(g[0], g[1], g[2]))
    u = jnp.stack(us)
    S = jnp.fft.rfftn(u, axes=(1, 2, 3))
    Sr = jnp.real(S).reshape(2, _ROWS, 128)
    Si = jnp.imag(S).reshape(2, _ROWS, 128)
    O = _scale_combine(Sr, Si)
    return O.reshape(2, 2, 128, 128, 65).transpose(2, 3, 4, 1, 0)


# speedup vs baseline: 1.5222x; 1.0232x over previous
import functools

import numpy as np
import jax
import jax.numpy as jnp
from jax import lax
from jax.experimental import pallas as pl
from jax.experimental.pallas import tpu as pltpu
from jax.experimental.pallas import tpu_sc as plsc

_RES = 128
_SIG = 10.0
_ROWS = 8320
_RCHUNK = 320


def _spec_consts():
    freqs = [np.fft.fftfreq(_RES, d=1.0 / _RES)] * 2
    freqs.append(np.fft.rfftfreq(_RES, d=1.0 / _RES))
    om = np.stack(np.meshgrid(*freqs, indexing="ij"), axis=-1)
    dis = np.sqrt((om ** 2).sum(-1))
    g = np.exp(-0.5 * ((_SIG * 2.0 * dis / _RES) ** 2))
    lap = -np.sum((2.0 * np.pi * om) ** 2, axis=-1)
    a = g / (lap + 1e-6)
    a[0, 0, 0] = 0.0
    return a.astype(np.float32).reshape(_ROWS, 128)


def _deriv_matrix():
    om = np.fft.fftfreq(_RES, d=1.0 / _RES)
    h = -2j * np.pi * om
    h[_RES // 2] = 0.0
    c = np.real(np.fft.ifft(h))
    idx = (np.arange(_RES)[:, None] - np.arange(_RES)[None, :]) % _RES
    return c[idx].astype(np.float32)


_A_CONST = _spec_consts()
_M_CONST = _deriv_matrix()


def _conv_div(g0, g1, g2):
    M = jnp.asarray(_M_CONST)
    MT = jnp.asarray(np.ascontiguousarray(_M_CONST.T))

    def body(g0_ref, g1_ref, g2_ref, m_ref, mt_ref, o_ref):
        j = pl.program_id(0)
        m = m_ref[...]
        mt = mt_ref[...]
        mrows = m_ref[pl.ds(j * 8, 8), :]
        g0r = g0_ref[...].reshape(_RES, _RES * _RES)
        xterm = jnp.dot(mrows, g0r, preferred_element_type=jnp.float32,
                        precision=jax.lax.Precision.HIGHEST)
        o_ref[...] = xterm.reshape(8, _RES, _RES)
        for p in range(8):
            yterm = jnp.dot(m, g1_ref[p], preferred_element_type=jnp.float32,
                            precision=jax.lax.Precision.HIGHEST)
            zterm = jnp.dot(g2_ref[p], mt, preferred_element_type=jnp.float32,
                            precision=jax.lax.Precision.HIGHEST)
            o_ref[p] += yterm + zterm

    return pl.pallas_call(
        body,
        grid=(_RES // 8,),
        in_specs=[
            pl.BlockSpec((_RES, _RES, _RES), lambda j: (0, 0, 0)),
            pl.BlockSpec((8, _RES, _RES), lambda j: (j, 0, 0)),
            pl.BlockSpec((8, _RES, _RES), lambda j: (j, 0, 0)),
            pl.BlockSpec((_RES, _RES), lambda j: (0, 0)),
            pl.BlockSpec((_RES, _RES), lambda j: (0, 0)),
        ],
        out_specs=pl.BlockSpec((8, _RES, _RES), lambda j: (j, 0, 0)),
        out_shape=jax.ShapeDtypeStruct((_RES, _RES, _RES), jnp.float32),
    )(g0, g1, g2, M, MT)


def _scale_combine(Sr, Si):
    A = jnp.asarray(_A_CONST)

    def body(sr_ref, si_ref, a_ref, o_ref):
        a = a_ref[...]
        for b in range(2):
            o_ref[2 * b] = a * sr_ref[b]
            o_ref[2 * b + 1] = a * si_ref[b]

    return pl.pallas_call(
        body,
        grid=(_ROWS // _RCHUNK,),
        in_specs=[
            pl.BlockSpec((2, _RCHUNK, 128), lambda i: (0, i, 0)),
            pl.BlockSpec((2, _RCHUNK, 128), lambda i: (0, i, 0)),
            pl.BlockSpec((_RCHUNK, 128), lambda i: (i, 0)),
        ],
        out_specs=pl.BlockSpec((4, _RCHUNK, 128), lambda i: (0, i, 0)),
        out_shape=jax.ShapeDtypeStruct((4, _ROWS, 128), jnp.float32),
    )(Sr, Si, A)


_P = 100352
_TPTS = _P // 16
_CHUNK = 1568
_NROW = _CHUNK // 16
_SLABW = 64
_SLAB = _SLABW * _RES * _RES
_DUMP = _SLAB
_ACC = _SLAB + 16384
_STRIPE = _SLAB // 16


def _make_sc_rasterize(b, f_feat):
    mesh = plsc.VectorSubcoreMesh(core_axis_name="c", subcore_axis_name="s")
    vbase = b * 3 * _P
    nbase = (b * 3 + f_feat) * _P

    @functools.partial(
        pl.kernel,
        out_type=jax.ShapeDtypeStruct((2 * _SLAB,), jnp.float32),
        mesh=mesh,
        scratch_types=[
            pltpu.VMEM_SHARED((_ACC,), jnp.float32),
            pltpu.VMEM((2048,), jnp.float32),
            pltpu.VMEM((_CHUNK,), jnp.float32),
            pltpu.VMEM((_CHUNK,), jnp.float32),
            pltpu.VMEM((_CHUNK,), jnp.float32),
            pltpu.VMEM((_CHUNK,), jnp.float32),
            pltpu.VMEM((_NROW * 128,), jnp.int32),
            pltpu.VMEM((_NROW * 128,), jnp.float32),
            pltpu.VMEM((_NROW * 128,), jnp.int32),
            pltpu.VMEM((_NROW * 128,), jnp.float32),
            pltpu.SemaphoreType.DMA,
            pltpu.SemaphoreType.DMA,
            pltpu.SemaphoreType.DMA,
        ],
    )
    def k(v_hbm, n_hbm, out_hbm, acc, zbuf, px, py, pz, nv,
          idxb0, valb0, idxb1, valb1, sem0, sem1, zsem):
        slab = lax.axis_index("c")
        s = lax.axis_index("s")

        def zinit(i, carry):
            zbuf[pl.ds(i * 16, 16)] = jnp.zeros((16,), jnp.float32)
            return carry

        lax.fori_loop(0, 2048 // 16, zinit, 0)

        zh = [pltpu.async_copy(zbuf,
                               acc.at[pl.ds(s * _STRIPE + i * 2048, 2048)],
                               zsem)
              for i in range(_STRIPE // 2048)]

        def fill(cc, idxb, valb):
            pbase = s * _TPTS + cc * _CHUNK
            pltpu.sync_copy(v_hbm.at[pl.ds(vbase + pbase, _CHUNK)], px)
            pltpu.sync_copy(v_hbm.at[pl.ds(vbase + _P + pbase, _CHUNK)], py)
            pltpu.sync_copy(v_hbm.at[pl.ds(vbase + 2 * _P + pbase, _CHUNK)], pz)
            pltpu.sync_copy(n_hbm.at[pl.ds(nbase + pbase, _CHUNK)], nv)

            def row(i, carry3):
                base = i * 16
                tx = px[pl.ds(base, 16)] * 128.0
                x0 = tx.astype(jnp.int32)
                fx = tx - x0.astype(jnp.float32)
                x1 = jnp.where(fx > 0.0, x0 + 1, x0) & 127
                ty = py[pl.ds(base, 16)] * 128.0
                y0 = ty.astype(jnp.int32)
                fy = ty - y0.astype(jnp.float32)
                y1 = jnp.where(fy > 0.0, y0 + 1, y0) & 127
                tz = pz[pl.ds(base, 16)] * 128.0
                z0 = tz.astype(jnp.int32)
                fz = tz - z0.astype(jnp.float32)
                z1 = jnp.where(fz > 0.0, z0 + 1, z0) & 127
                val = nv[pl.ds(base, 16)]

                dump = jnp.full((16,), _DUMP, jnp.int32)
                xo0 = jnp.where((x0 >> 6) == slab, (x0 & 63) * 16384, dump)
                xo1 = jnp.where((x1 >> 6) == slab, (x1 & 63) * 16384, dump)
                a00 = xo0 + y0 * 128
                a01 = xo0 + y1 * 128
                a10 = xo1 + y0 * 128
                a11 = xo1 + y1 * 128
                wx0 = 1.0 - fx
                wy0 = 1.0 - fy
                wz0 = (1.0 - fz) * val
                wz1 = fz * val
                w00 = wx0 * wy0
                w01 = wx0 * fy
                w10 = fx * wy0
                w11 = fx * fy
                idxb[pl.ds(i * 128 + 0, 16)] = a00 + z0
                valb[pl.ds(i * 128 + 0, 16)] = w00 * wz0
                idxb[pl.ds(i * 128 + 16, 16)] = a00 + z1
                valb[pl.ds(i * 128 + 16, 16)] = w00 * wz1
                idxb[pl.ds(i * 128 + 32, 16)] = a01 + z0
                valb[pl.ds(i * 128 + 32, 16)] = w01 * wz0
                idxb[pl.ds(i * 128 + 48, 16)] = a01 + z1
                valb[pl.ds(i * 128 + 48, 16)] = w01 * wz1
                idxb[pl.ds(i * 128 + 64, 16)] = a10 + z0
                valb[pl.ds(i * 128 + 64, 16)] = w10 * wz0
                idxb[pl.ds(i * 128 + 80, 16)] = a10 + z1
                valb[pl.ds(i * 128 + 80, 16)] = w10 * wz1
                idxb[pl.ds(i * 128 + 96, 16)] = a11 + z0
                valb[pl.ds(i * 128 + 96, 16)] = w11 * wz0
                idxb[pl.ds(i * 128 + 112, 16)] = a11 + z1
                valb[pl.ds(i * 128 + 112, 16)] = w11 * wz1
                return carry3

            lax.fori_loop(0, _NROW, row, 0)

        def chunk(cc, idxb, valb, sem):
            fill(cc, idxb, valb)
            return pltpu.async_copy(valb, acc.at[idxb], sem, add=True)

        bufs = ((idxb0, valb0, sem0), (idxb1, valb1, sem1))
        handles = [None, None]
        for cc in range(_TPTS // _CHUNK):
            sel = cc % 2
            if handles[sel] is not None:
                handles[sel].wait()
            if cc == 0:
                fill(0, *bufs[0][:2])
                for h in zh:
                    h.wait()
                plsc.subcore_barrier()
                handles[0] = pltpu.async_copy(bufs[0][1], acc.at[bufs[0][0]],
                                              bufs[0][2], add=True)
            else:
                handles[sel] = chunk(cc, *bufs[sel])
        for h in handles:
            h.wait()
        plsc.subcore_barrier()

        pltpu.sync_copy(
            acc.at[pl.ds(s * _STRIPE, _STRIPE)],
            out_hbm.at[pl.ds(slab * _SLAB + s * _STRIPE, _STRIPE)])

    return k


_SC_RASTER = [[_make_sc_rasterize(b, f) for f in range(3)] for b in range(2)]


def kernel(V, N):
    npts = V.shape[1]
    Vt = jnp.pad(jnp.transpose(V, (0, 2, 1)),
                 ((0, 0), (0, 0), (0, _P - npts))).reshape(-1)
    Nt = jnp.pad(jnp.transpose(N, (0, 2, 1)),
                 ((0, 0), (0, 0), (0, _P - npts))).reshape(-1)
    us = []
    for b in range(2):
        g = [_SC_RASTER[b][f](Vt, Nt).reshape(_RES, _RES, _RES)
             for f in range(3)]
        us.append(_conv_div(g[0], g[1], g[2]))
    u = jnp.stack(us)
    S = jnp.fft.rfftn(u, axes=(1, 2, 3))
    Sr = jnp.real(S).reshape(2, _ROWS, 128)
    Si = jnp.imag(S).reshape(2, _ROWS, 128)
    O = _scale_combine(Sr, Si)
    return O.reshape(2, 2, 128, 128, 65).transpose(2, 3, 4, 1, 0)
